# baseline (speedup 1.0000x reference)
import functools

import jax
import jax.numpy as jnp
from jax import lax
from jax.experimental import pallas as pl
from jax.experimental.pallas import tpu as pltpu
from jax.experimental.pallas import tpu_sc as plsc

B = 4096
S = 200
E = 64
VOCAB = 1000000

NC = 2
NS = 16
NW = NC * NS
B_PER_W = B // NW
G = 8
HALF = S // 2


NCH = 2 * B_PER_W
K = 4
VB = 4096
EP = 128
HALFV = 512000
NBLK = HALFV // VB
CW = 112
GHALF = 104


def _transpose_body(t_ref, u_ref, o_ref):
    a = t_ref[...]
    b = u_ref[...]
    o_ref[...] = jnp.concatenate([a.T, b.T], axis=1)


def _transpose(tT):
    return pl.pallas_call(
        _transpose_body,
        grid=(NBLK,),
        in_specs=[
            pl.BlockSpec((E, VB), lambda k: (0, k)),
            pl.BlockSpec((E, VB), lambda k: (0, jnp.minimum(NBLK + k, 244))),
        ],
        out_specs=pl.BlockSpec((VB, EP), lambda k: (k, 0)),
        out_shape=jax.ShapeDtypeStruct((HALFV, EP), jnp.float32),
    )(tT, tT)


def _pool_body(x2_hbm, table_hbm, out_hbm, idx_v, off_v, buf_v, out_v,
               s0, s1, s2, s3):
    wid = lax.axis_index("s") * NC + lax.axis_index("c")
    base = wid * B_PER_W
    sems = (s0, s1, s2, s3)

    pltpu.sync_copy(x2_hbm.at[pl.ds(base * 2, NCH)], idx_v)

    def prep(j, carry):
        for g in range(CW // 16):
            v = idx_v[j, pl.ds(g * 16, 16)]
            m = v >= HALFV
            idx_v[j, pl.ds(g * 16, 16)] = jnp.where(m, v - HALFV, v)
            off_v[j, pl.ds(g * 16, 16)] = jnp.where(m, E, 0)
        return carry

    lax.fori_loop(0, NCH, prep, 0)

    for k in range(K):
        pltpu.async_copy(
            table_hbm.at[idx_v.at[k, pl.ds(0, GHALF)]], buf_v.at[k], sems[k]
        )

    def body(rr, carry):
        r0 = rr * 2
        for u in range(2):
            r = r0 + u
            accs = (jnp.zeros((16,), jnp.float32),) * 4
            for h in range(2):
                k = 2 * u + h
                j = 2 * r + h
                pltpu.make_async_copy(
                    table_hbm.at[idx_v.at[j, pl.ds(0, GHALF)]],
                    buf_v.at[k], sems[k],
                ).wait()

                def red(g, accs, j=j, k=k, n=16):
                    offv = off_v[j, pl.ds(g * 16, 16)]
                    for l in range(n):
                        off = offv[l]
                        accs = tuple(
                            accs[c] + buf_v[k, g * 16 + l, pl.ds(off + c * 16, 16)]
                            for c in range(4)
                        )
                    return accs

                accs = lax.fori_loop(0, HALF // 16, red, accs)
                accs = red(HALF // 16, accs, n=HALF % 16)

                @pl.when(j + K < NCH)
                def _(j=j, k=k):
                    pltpu.async_copy(
                        table_hbm.at[idx_v.at[j + K, pl.ds(0, GHALF)]],
                        buf_v.at[k], sems[k],
                    )

            for c in range(4):
                out_v[r, pl.ds(c * 16, 16)] = accs[c] * (1.0 / S)
        return carry

    lax.fori_loop(0, B_PER_W // 2, body, 0)
    pltpu.sync_copy(out_v, out_hbm.at[pl.ds(base, B_PER_W)])


def _pool(x2, table):
    mesh = plsc.VectorSubcoreMesh(core_axis_name="c", subcore_axis_name="s")
    fn = functools.partial(
        pl.kernel,
        out_type=jax.ShapeDtypeStruct((B, E), jnp.float32),
        mesh=mesh,
        scratch_types=[
            pltpu.VMEM((NCH, CW), jnp.int32),
            pltpu.VMEM((NCH, CW), jnp.int32),
            pltpu.VMEM((K, GHALF, EP), jnp.float32),
            pltpu.VMEM((B_PER_W, E), jnp.float32),
            pltpu.SemaphoreType.DMA,
            pltpu.SemaphoreType.DMA,
            pltpu.SemaphoreType.DMA,
            pltpu.SemaphoreType.DMA,
        ],
        compiler_params=pltpu.CompilerParams(use_tc_tiling_on_sc=False),
    )(_pool_body)
    return fn(x2, table)


def _mlp_body(p_ref, w1, b1, w2, b2, w3, b3, o_ref):
    y = jnp.dot(p_ref[...], w1[...], preferred_element_type=jnp.float32)
    y = jnp.maximum(y + b1[...], 0.0)
    y = jnp.dot(y, w2[...], preferred_element_type=jnp.float32)
    y = jnp.maximum(y + b2[...], 0.0)
    o_ref[...] = jnp.dot(y, w3[...], preferred_element_type=jnp.float32) + b3[...]


def _mlp(pooled, W1, b1, W2, b2, W3, b3):
    nc = W3.shape[1]
    return pl.pallas_call(
        _mlp_body,
        out_shape=jax.ShapeDtypeStruct((B, nc), jnp.float32),
    )(pooled, W1, b1.reshape(1, -1), W2, b2.reshape(1, -1), W3, b3.reshape(1, -1))


@jax.jit
def kernel(x, table, W1, b1, W2, b2, W3, b3):
    x2 = jnp.pad(x.reshape(B * 2, HALF), ((0, 0), (0, CW - HALF)))
    t2 = _transpose(table.T)
    pooled = _pool(x2, t2)
    return _mlp(pooled, W1, b1, W2, b2, W3, b3)


# speedup vs baseline: 3.8299x; 3.8299x over previous
import functools

import jax
import jax.numpy as jnp
from jax import lax
from jax.experimental import pallas as pl
from jax.experimental.pallas import tpu as pltpu
from jax.experimental.pallas import tpu_sc as plsc

B = 4096
S = 200
E = 64
VOCAB = 1000000

NC = 2
NS = 16
NW = NC * NS
B_PER_W = B // NW
G = 8
HALF = S // 2


NCH = 2 * B_PER_W
K = 4
VB = 8192
EP = 128


def _transpose_body(t_ref, o_ref):
    a = t_ref[...]
    o_ref[:, pl.ds(0, E)] = a.T


def _transpose(tT):
    grid = (VOCAB + VB - 1) // VB
    return pl.pallas_call(
        _transpose_body,
        grid=(grid,),
        in_specs=[pl.BlockSpec((E, VB), lambda k: (0, k))],
        out_specs=pl.BlockSpec((VB, EP), lambda k: (k, 0)),
        out_shape=jax.ShapeDtypeStruct((VOCAB, EP), jnp.float32),
    )(tT)


def _pool_body(x2_hbm, table_hbm, out_hbm, idx_v, buf_v, out_v, s0, s1, s2, s3):
    wid = lax.axis_index("s") * NC + lax.axis_index("c")
    base = wid * B_PER_W
    sems = (s0, s1, s2, s3)

    pltpu.sync_copy(x2_hbm.at[pl.ds(base * 2, NCH)], idx_v)

    for k in range(K):
        pltpu.async_copy(table_hbm.at[idx_v.at[k]], buf_v.at[k], sems[k])

    def body(rr, carry):
        r0 = rr * 2
        for u in range(2):
            r = r0 + u
            accs = (jnp.zeros((16,), jnp.float32),) * 4
            for h in range(2):
                k = 2 * u + h
                j = 2 * r + h
                pltpu.make_async_copy(
                    table_hbm.at[idx_v.at[j]], buf_v.at[k], sems[k]
                ).wait()

                def red(i, accs, k=k):
                    return tuple(
                        accs[c] + buf_v[k, i, pl.ds(c * 16, 16)]
                        for c in range(4)
                    )

                accs = lax.fori_loop(0, HALF, red, accs)

                @pl.when(j + K < NCH)
                def _(j=j, k=k):
                    pltpu.async_copy(
                        table_hbm.at[idx_v.at[j + K]], buf_v.at[k], sems[k]
                    )

            for c in range(4):
                out_v[r, pl.ds(c * 16, 16)] = accs[c] * (1.0 / S)
        return carry

    lax.fori_loop(0, B_PER_W // 2, body, 0)
    pltpu.sync_copy(out_v, out_hbm.at[pl.ds(base, B_PER_W)])


def _pool(x2, table):
    mesh = plsc.VectorSubcoreMesh(core_axis_name="c", subcore_axis_name="s")
    fn = functools.partial(
        pl.kernel,
        out_type=jax.ShapeDtypeStruct((B, E), jnp.float32),
        mesh=mesh,
        scratch_types=[
            pltpu.VMEM((NCH, HALF), jnp.int32),
            pltpu.VMEM((K, HALF, EP), jnp.float32),
            pltpu.VMEM((B_PER_W, E), jnp.float32),
            pltpu.SemaphoreType.DMA,
            pltpu.SemaphoreType.DMA,
            pltpu.SemaphoreType.DMA,
            pltpu.SemaphoreType.DMA,
        ],
        compiler_params=pltpu.CompilerParams(use_tc_tiling_on_sc=False),
    )(_pool_body)
    return fn(x2, table)


def _mlp_body(p_ref, w1, b1, w2, b2, w3, b3, o_ref):
    y = jnp.dot(p_ref[...], w1[...], preferred_element_type=jnp.float32)
    y = jnp.maximum(y + b1[...], 0.0)
    y = jnp.dot(y, w2[...], preferred_element_type=jnp.float32)
    y = jnp.maximum(y + b2[...], 0.0)
    o_ref[...] = jnp.dot(y, w3[...], preferred_element_type=jnp.float32) + b3[...]


def _mlp(pooled, W1, b1, W2, b2, W3, b3):
    nc = W3.shape[1]
    return pl.pallas_call(
        _mlp_body,
        out_shape=jax.ShapeDtypeStruct((B, nc), jnp.float32),
    )(pooled, W1, b1.reshape(1, -1), W2, b2.reshape(1, -1), W3, b3.reshape(1, -1))


@jax.jit
def kernel(x, table, W1, b1, W2, b2, W3, b3):
    x2 = x.reshape(B * 2, HALF)
    t2 = _transpose(table.T)
    pooled = _pool(x2, t2)
    return _mlp(pooled, W1, b1, W2, b2, W3, b3)


# speedup vs baseline: 3.9956x; 1.0433x over previous
import functools

import jax
import jax.numpy as jnp
from jax import lax
from jax.experimental import pallas as pl
from jax.experimental.pallas import tpu as pltpu
from jax.experimental.pallas import tpu_sc as plsc

B = 4096
S = 200
E = 64
VOCAB = 1000000

NC = 2
NS = 16
NW = NC * NS
B_PER_W = B // NW
G = 8
HALF = S // 2


NCH = 2 * B_PER_W
K = 4
VB = 16384
EP = 128


def _transpose_body(t_ref, o_ref):
    a = t_ref[...]
    o_ref[:, pl.ds(0, E)] = a.T


def _transpose(tT):
    grid = (VOCAB + VB - 1) // VB
    return pl.pallas_call(
        _transpose_body,
        grid=(grid,),
        in_specs=[pl.BlockSpec((E, VB), lambda k: (0, k))],
        out_specs=pl.BlockSpec((VB, EP), lambda k: (k, 0)),
        out_shape=jax.ShapeDtypeStruct((VOCAB, EP), jnp.float32),
    )(tT)


def _pool_body(x2_hbm, table_hbm, out_hbm, idx_v, buf_v, out_v, s0, s1, s2, s3):
    wid = lax.axis_index("s") * NC + lax.axis_index("c")
    base = wid * B_PER_W
    sems = (s0, s1, s2, s3)

    pltpu.sync_copy(x2_hbm.at[pl.ds(base * 2, NCH)], idx_v)

    for k in range(K):
        pltpu.async_copy(table_hbm.at[idx_v.at[k]], buf_v.at[k], sems[k])

    def body(rr, carry):
        r0 = rr * 2
        for u in range(2):
            r = r0 + u
            accs = (jnp.zeros((16,), jnp.float32),) * 4
            for h in range(2):
                k = 2 * u + h
                j = 2 * r + h
                pltpu.make_async_copy(
                    table_hbm.at[idx_v.at[j]], buf_v.at[k], sems[k]
                ).wait()

                def red(i, accs, k=k):
                    return tuple(
                        accs[c] + buf_v[k, i, pl.ds(c * 16, 16)]
                        for c in range(4)
                    )

                accs = lax.fori_loop(0, HALF, red, accs)

                @pl.when(j + K < NCH)
                def _(j=j, k=k):
                    pltpu.async_copy(
                        table_hbm.at[idx_v.at[j + K]], buf_v.at[k], sems[k]
                    )

            for c in range(4):
                out_v[r, pl.ds(c * 16, 16)] = accs[c] * (1.0 / S)
        return carry

    lax.fori_loop(0, B_PER_W // 2, body, 0)
    pltpu.sync_copy(out_v, out_hbm.at[pl.ds(base, B_PER_W)])


def _pool(x2, table):
    mesh = plsc.VectorSubcoreMesh(core_axis_name="c", subcore_axis_name="s")
    fn = functools.partial(
        pl.kernel,
        out_type=jax.ShapeDtypeStruct((B, E), jnp.float32),
        mesh=mesh,
        scratch_types=[
            pltpu.VMEM((NCH, HALF), jnp.int32),
            pltpu.VMEM((K, HALF, EP), jnp.float32),
            pltpu.VMEM((B_PER_W, E), jnp.float32),
            pltpu.SemaphoreType.DMA,
            pltpu.SemaphoreType.DMA,
            pltpu.SemaphoreType.DMA,
            pltpu.SemaphoreType.DMA,
        ],
        compiler_params=pltpu.CompilerParams(use_tc_tiling_on_sc=False),
    )(_pool_body)
    return fn(x2, table)


def _mlp_body(p_ref, w1, b1, w2, b2, w3, b3, o_ref):
    y = jnp.dot(p_ref[...], w1[...], preferred_element_type=jnp.float32)
    y = jnp.maximum(y + b1[...], 0.0)
    y = jnp.dot(y, w2[...], preferred_element_type=jnp.float32)
    y = jnp.maximum(y + b2[...], 0.0)
    o_ref[...] = jnp.dot(y, w3[...], preferred_element_type=jnp.float32) + b3[...]


def _mlp(pooled, W1, b1, W2, b2, W3, b3):
    nc = W3.shape[1]
    return pl.pallas_call(
        _mlp_body,
        out_shape=jax.ShapeDtypeStruct((B, nc), jnp.float32),
    )(pooled, W1, b1.reshape(1, -1), W2, b2.reshape(1, -1), W3, b3.reshape(1, -1))


@jax.jit
def kernel(x, table, W1, b1, W2, b2, W3, b3):
    x2 = x.reshape(B * 2, HALF)
    t2 = _transpose(table.T)
    pooled = _pool(x2, t2)
    return _mlp(pooled, W1, b1, W2, b2, W3, b3)


# speedup vs baseline: 4.5470x; 1.1380x over previous
import functools

import jax
import jax.numpy as jnp
from jax import lax
from jax.experimental import pallas as pl
from jax.experimental.pallas import tpu as pltpu
from jax.experimental.pallas import tpu_sc as plsc

B = 4096
S = 200
E = 64
VOCAB = 1000000

NC = 2
NS = 16
NW = NC * NS
B_PER_W = B // NW
G = 8
HALF = S // 2


NCH = 2 * B_PER_W
K = 4
VB = 16384
EP = 128


def _transpose_body(t_ref, o_ref):
    a = t_ref[...]
    o_ref[:, pl.ds(0, E)] = a.T


def _transpose(tT):
    grid = (VOCAB + VB - 1) // VB
    return pl.pallas_call(
        _transpose_body,
        grid=(grid,),
        in_specs=[pl.BlockSpec((E, VB), lambda k: (0, k))],
        out_specs=pl.BlockSpec((VB, EP), lambda k: (k, 0)),
        out_shape=jax.ShapeDtypeStruct((VOCAB, EP), jnp.float32),
    )(tT)


def _pool_body(x2_hbm, table_hbm, out_hbm, idx_v, buf_v, out_v, s0, s1, s2, s3):
    wid = lax.axis_index("s") * NC + lax.axis_index("c")
    base = wid * B_PER_W
    sems = (s0, s1, s2, s3)

    pltpu.sync_copy(x2_hbm.at[pl.ds(base * 2, NCH)], idx_v)

    for k in range(K):
        pltpu.async_copy(table_hbm.at[idx_v.at[k]], buf_v.at[k], sems[k])

    def body(rr, carry):
        r0 = rr * 2
        for u in range(2):
            r = r0 + u
            accs = (jnp.zeros((16,), jnp.float32),) * 4
            for h in range(2):
                k = 2 * u + h
                j = 2 * r + h
                pltpu.make_async_copy(
                    table_hbm.at[idx_v.at[j]], buf_v.at[k], sems[k]
                ).wait()

                def red(i, accs, k=k):
                    return tuple(
                        accs[c] + buf_v[k, i, pl.ds(c * 16, 16)]
                        for c in range(4)
                    )

                accs = lax.fori_loop(0, HALF, red, accs)

                @pl.when(j + K < NCH)
                def _(j=j, k=k):
                    pltpu.async_copy(
                        table_hbm.at[idx_v.at[j + K]], buf_v.at[k], sems[k]
                    )

            for c in range(4):
                out_v[r, pl.ds(c * 16, 16)] = accs[c] * (1.0 / S)
        return carry

    lax.fori_loop(0, B_PER_W // 2, body, 0)
    pltpu.sync_copy(out_v, out_hbm.at[pl.ds(base, B_PER_W)])


def _pool(x2, table):
    mesh = plsc.VectorSubcoreMesh(core_axis_name="c", subcore_axis_name="s")
    fn = functools.partial(
        pl.kernel,
        out_type=jax.ShapeDtypeStruct((B, E), jnp.float32),
        mesh=mesh,
        scratch_types=[
            pltpu.VMEM((NCH, HALF), jnp.int32),
            pltpu.VMEM((K, HALF, E), jnp.float32),
            pltpu.VMEM((B_PER_W, E), jnp.float32),
            pltpu.SemaphoreType.DMA,
            pltpu.SemaphoreType.DMA,
            pltpu.SemaphoreType.DMA,
            pltpu.SemaphoreType.DMA,
        ],
        compiler_params=pltpu.CompilerParams(use_tc_tiling_on_sc=False),
    )(_pool_body)
    return fn(x2, table)


def _mlp_body(p_ref, w1, b1, w2, b2, w3, b3, o_ref):
    y = jnp.dot(p_ref[...], w1[...], preferred_element_type=jnp.float32)
    y = jnp.maximum(y + b1[...], 0.0)
    y = jnp.dot(y, w2[...], preferred_element_type=jnp.float32)
    y = jnp.maximum(y + b2[...], 0.0)
    o_ref[...] = jnp.dot(y, w3[...], preferred_element_type=jnp.float32) + b3[...]


def _mlp(pooled, W1, b1, W2, b2, W3, b3):
    nc = W3.shape[1]
    return pl.pallas_call(
        _mlp_body,
        out_shape=jax.ShapeDtypeStruct((B, nc), jnp.float32),
    )(pooled, W1, b1.reshape(1, -1), W2, b2.reshape(1, -1), W3, b3.reshape(1, -1))


@jax.jit
def kernel(x, table, W1, b1, W2, b2, W3, b3):
    x2 = x.reshape(B * 2, HALF) * 2
    t2 = _transpose(table.T).reshape(2 * VOCAB, E)
    pooled = _pool(x2, t2)
    return _mlp(pooled, W1, b1, W2, b2, W3, b3)
